# Initial kernel scaffold; baseline (speedup 1.0000x reference)
import functools

import jax
import jax.numpy as jnp
from jax import lax
from jax.experimental import pallas as pl
from jax.experimental.pallas import tpu as pltpu
from jax.experimental.pallas import tpu_sc as plsc

NC = 2
NS = 16
NW = NC * NS
CH = 128
CW = 16
ZR = 500


def _mesh():
    return plsc.VectorSubcoreMesh(core_axis_name="c", subcore_axis_name="s",
                                  num_cores=NC, num_subcores=NS)


def _sc_gather(table, src, etype, norm, att):
    n, d = table.shape
    e = src.shape[0]
    r, nb = att.shape
    nch = e // CH
    jmax = (nch + NW - 1) // NW

    @functools.partial(
        pl.kernel,
        out_type=(jax.ShapeDtypeStruct((e, d), jnp.float32),
                  jax.ShapeDtypeStruct((e, nb), jnp.float32)),
        mesh=_mesh(),
        scratch_types=[
            pltpu.VMEM((r, nb), jnp.float32),
            pltpu.VMEM((CH,), jnp.int32),
            pltpu.VMEM((CH,), jnp.int32),
            pltpu.VMEM((CH,), jnp.float32),
            pltpu.VMEM((CH, d), jnp.float32),
            pltpu.VMEM((CH, nb), jnp.float32),
            pltpu.SemaphoreType.DMA,
        ],
    )
    def k(table_h, src_h, et_h, norm_h, att_h, xe_h, a_h,
          att_v, sidx, tbuf, nbuf, xrows, abuf, sem):
        c = lax.axis_index("c")
        s = lax.axis_index("s")
        w = s * NC + c
        pltpu.sync_copy(att_h, att_v)

        def body(j, carry):
            kk = w + NW * j

            @pl.when(kk < nch)
            def _():
                base = kk * CH
                pltpu.sync_copy(src_h.at[pl.ds(base, CH)], sidx)
                cp = pltpu.async_copy(table_h.at[sidx], xrows, sem)
                pltpu.sync_copy(et_h.at[pl.ds(base, CH)], tbuf)
                pltpu.sync_copy(norm_h.at[pl.ds(base, CH)], nbuf)
                for g in range(CH // 16):
                    t16 = tbuf[pl.ds(g * 16, 16)]
                    n16 = nbuf[pl.ds(g * 16, 16)]
                    eidx = lax.iota(jnp.int32, 16) + g * 16
                    for b in range(nb):
                        bfull = jnp.full((16,), b, jnp.int32)
                        av = plsc.load_gather(att_v, [t16, bfull])
                        plsc.store_scatter(abuf, [eidx, bfull], av * n16)
                cp.wait()
                pltpu.sync_copy(xrows, xe_h.at[pl.ds(base, CH)])
                pltpu.sync_copy(abuf, a_h.at[pl.ds(base, CH)])

            return carry

        lax.fori_loop(0, jmax, body, jnp.int32(0))

    return k(table, src, etype, norm, att)


def _sc_scatter(msg, dst, n, with_count):
    e, d = msg.shape
    nch = e // CH
    jmax = (nch + NW - 1) // NW
    rows_per_sub = n // NS
    nz = rows_per_sub // ZR

    out_type = [jax.ShapeDtypeStruct((NC, n, d), jnp.float32)]
    scratch = [
        pltpu.VMEM_SHARED((n, d), jnp.float32),
        pltpu.VMEM((CH,), jnp.int32),
        pltpu.VMEM((CH, d), jnp.float32),
        pltpu.VMEM((ZR, d), jnp.float32),
    ]
    if with_count:
        out_type.append(jax.ShapeDtypeStruct((NC, n, CW), jnp.float32))
        scratch += [
            pltpu.VMEM_SHARED((n, CW), jnp.float32),
            pltpu.VMEM((ZR, CW), jnp.float32),
            pltpu.VMEM((CH, CW), jnp.float32),
        ]

    @functools.partial(pl.kernel, out_type=tuple(out_type), mesh=_mesh(),
                       scratch_types=scratch)
    def k(msg_h, dst_h, *refs):
        if with_count:
            agg_h, cnt_h, agg_sh, didx, mbuf, zbuf, cnt_sh, zbuf2, ones = refs
        else:
            agg_h, agg_sh, didx, mbuf, zbuf = refs
        c = lax.axis_index("c")
        s = lax.axis_index("s")
        w = s * NC + c
        sub_base = s * rows_per_sub

        z16 = jnp.zeros((16,), jnp.float32)
        o16 = jnp.ones((16,), jnp.float32)

        def zfill(i, carry):
            for col in range(0, d, 16):
                zbuf[i, pl.ds(col, 16)] = z16
            if with_count:
                for col in range(0, CW, 16):
                    zbuf2[i, pl.ds(col, 16)] = z16
            return carry

        lax.fori_loop(0, ZR, zfill, jnp.int32(0))
        if with_count:
            def ofill(i, carry):
                for col in range(0, CW, 16):
                    ones[i, pl.ds(col, 16)] = o16
                return carry
            lax.fori_loop(0, CH, ofill, jnp.int32(0))

        for q in range(nz):
            pltpu.sync_copy(zbuf, agg_sh.at[pl.ds(sub_base + q * ZR, ZR)])
            if with_count:
                pltpu.sync_copy(zbuf2, cnt_sh.at[pl.ds(sub_base + q * ZR, ZR)])
        plsc.subcore_barrier()

        def body(j, carry):
            kk = w + NW * j

            @pl.when(kk < nch)
            def _():
                base = kk * CH
                pltpu.sync_copy(dst_h.at[pl.ds(base, CH)], didx)
                pltpu.sync_copy(msg_h.at[pl.ds(base, CH)], mbuf)
                pltpu.sync_copy(mbuf, agg_sh.at[didx], add=True)
                if with_count:
                    pltpu.sync_copy(ones, cnt_sh.at[didx], add=True)

            return carry

        lax.fori_loop(0, jmax, body, jnp.int32(0))
        plsc.subcore_barrier()

        pltpu.sync_copy(agg_sh.at[pl.ds(sub_base, rows_per_sub)],
                        agg_h.at[c, pl.ds(sub_base, rows_per_sub)])
        if with_count:
            pltpu.sync_copy(cnt_sh.at[pl.ds(sub_base, rows_per_sub)],
                            cnt_h.at[c, pl.ds(sub_base, rows_per_sub)])

    return k(msg, dst)


def _tc_contract(xe, a, bmat):
    e, d = xe.shape
    nb = a.shape[1]
    be = 1600
    grid = e // be

    def body(xe_ref, a_ref, bm_ref, out_ref):
        y = jnp.dot(xe_ref[...], bm_ref[...], preferred_element_type=jnp.float32)
        av = a_ref[...]
        acc = av[:, 0:1] * y[:, 0:d]
        for b in range(1, nb):
            acc = acc + av[:, b:b + 1] * y[:, b * d:(b + 1) * d]
        out_ref[...] = acc

    return pl.pallas_call(
        body,
        grid=(grid,),
        in_specs=[
            pl.BlockSpec((be, d), lambda i: (i, 0)),
            pl.BlockSpec((be, nb), lambda i: (i, 0)),
            pl.BlockSpec(bmat.shape, lambda i: (0, 0)),
        ],
        out_specs=pl.BlockSpec((be, d), lambda i: (i, 0)),
        out_shape=jax.ShapeDtypeStruct((e, d), jnp.float32),
    )(xe, a, bmat)


def _tc_finish(agg, cnt_or_inv, x, root, bias, first_layer):
    n, d = x.shape
    bn = 2000
    grid = n // bn

    if first_layer:
        def body(agg_ref, cnt_ref, x_ref, root_ref, bias_ref, h_ref, inv_ref):
            cc = cnt_ref[0, :, 0:1] + cnt_ref[1, :, 0:1]
            inv = 1.0 / jnp.maximum(cc, 1.0)
            aggs = agg_ref[0] + agg_ref[1]
            h = aggs * inv + jnp.dot(x_ref[...], root_ref[...],
                                     preferred_element_type=jnp.float32)
            h = h + bias_ref[...]
            h_ref[...] = jnp.maximum(h, 0.0)
            inv_ref[...] = inv

        return pl.pallas_call(
            body,
            grid=(grid,),
            in_specs=[
                pl.BlockSpec((NC, bn, d), lambda i: (0, i, 0)),
                pl.BlockSpec((NC, bn, CW), lambda i: (0, i, 0)),
                pl.BlockSpec((bn, d), lambda i: (i, 0)),
                pl.BlockSpec((d, d), lambda i: (0, 0)),
                pl.BlockSpec((1, d), lambda i: (0, 0)),
            ],
            out_specs=[
                pl.BlockSpec((bn, d), lambda i: (i, 0)),
                pl.BlockSpec((bn, 1), lambda i: (i, 0)),
            ],
            out_shape=[jax.ShapeDtypeStruct((n, d), jnp.float32),
                       jax.ShapeDtypeStruct((n, 1), jnp.float32)],
        )(agg, cnt_or_inv, x, root, bias)

    def body(agg_ref, inv_ref, x_ref, root_ref, bias_ref, out_ref):
        aggs = agg_ref[0] + agg_ref[1]
        h = aggs * inv_ref[...] + jnp.dot(x_ref[...], root_ref[...],
                                          preferred_element_type=jnp.float32)
        out_ref[...] = h + bias_ref[...]

    return pl.pallas_call(
        body,
        grid=(grid,),
        in_specs=[
            pl.BlockSpec((NC, bn, d), lambda i: (0, i, 0)),
            pl.BlockSpec((bn, 1), lambda i: (i, 0)),
            pl.BlockSpec((bn, d), lambda i: (i, 0)),
            pl.BlockSpec((d, d), lambda i: (0, 0)),
            pl.BlockSpec((1, d), lambda i: (0, 0)),
        ],
        out_specs=pl.BlockSpec((bn, d), lambda i: (i, 0)),
        out_shape=jax.ShapeDtypeStruct((n, d), jnp.float32),
    )(agg, cnt_or_inv, x, root, bias)


def kernel(entity, edge_index, edge_type, edge_norm, emb_table,
           basis1, att1, root1, bias1, basis2, att2, root2, bias2):
    n, d = emb_table.shape
    nb = basis1.shape[0]
    x = emb_table
    src = edge_index[0]
    dst = edge_index[1]
    bmat1 = basis1.transpose(1, 0, 2).reshape(d, nb * d)
    bmat2 = basis2.transpose(1, 0, 2).reshape(d, nb * d)

    xe1, a1 = _sc_gather(x, src, edge_type, edge_norm, att1)
    msg1 = _tc_contract(xe1, a1, bmat1)
    agg1, cnt = _sc_scatter(msg1, dst, n, with_count=True)
    h, inv = _tc_finish(agg1, cnt, x, root1, bias1.reshape(1, d), first_layer=True)

    xe2, a2 = _sc_gather(h, src, edge_type, edge_norm, att2)
    msg2 = _tc_contract(xe2, a2, bmat2)
    agg2 = _sc_scatter(msg2, dst, n, with_count=False)
    out = _tc_finish(agg2, inv, h, root2, bias2.reshape(1, d), first_layer=False)
    return out


# speedup vs baseline: 1.9739x; 1.9739x over previous
import functools

import jax
import jax.numpy as jnp
from jax import lax
from jax.experimental import pallas as pl
from jax.experimental.pallas import tpu as pltpu
from jax.experimental.pallas import tpu_sc as plsc

NC = 2
NS = 16
NW = NC * NS
CH = 128
CW = 8
ZR = 160


def _mesh():
    return plsc.VectorSubcoreMesh(core_axis_name="c", subcore_axis_name="s",
                                  num_cores=NC, num_subcores=NS)


def _sc_gather(table, src, etype, norm, att):
    n, d = table.shape
    e = src.shape[0]
    r, nb = att.shape
    nch = e // CH
    jmax = (nch + NW - 1) // NW

    @functools.partial(
        pl.kernel,
        out_type=(jax.ShapeDtypeStruct((e, d), jnp.float32),
                  jax.ShapeDtypeStruct((e * nb,), jnp.float32)),
        mesh=_mesh(),
        scratch_types=[
            pltpu.VMEM((r * nb,), jnp.float32),
            pltpu.VMEM((CH,), jnp.int32),
            pltpu.VMEM((CH,), jnp.int32),
            pltpu.VMEM((CH,), jnp.float32),
            pltpu.VMEM((CH, d), jnp.float32),
            pltpu.VMEM((CH * nb,), jnp.float32),
            pltpu.SemaphoreType.DMA,
        ],
        compiler_params=pltpu.CompilerParams(needs_layout_passes=False,
                                             use_tc_tiling_on_sc=False),
    )
    def k(table_h, src_h, et_h, norm_h, att_h, xe_h, a_h,
          att_v, sidx, tbuf, nbuf, xrows, abuf, sem):
        c = lax.axis_index("c")
        s = lax.axis_index("s")
        w = s * NC + c
        pltpu.sync_copy(att_h, att_v)

        def body(j, carry):
            kk = w + NW * j

            @pl.when(kk < nch)
            def _():
                base = kk * CH
                pltpu.sync_copy(src_h.at[pl.ds(base, CH)], sidx)
                cp = pltpu.async_copy(table_h.at[sidx], xrows, sem)
                pltpu.sync_copy(et_h.at[pl.ds(base, CH)], tbuf)
                pltpu.sync_copy(norm_h.at[pl.ds(base, CH)], nbuf)
                for g in range(CH // 16):
                    t16 = tbuf[pl.ds(g * 16, 16)] * nb
                    n16 = nbuf[pl.ds(g * 16, 16)]
                    eidx = (lax.iota(jnp.int32, 16) + g * 16) * nb
                    for b in range(nb):
                        av = plsc.load_gather(att_v, [t16 + b])
                        plsc.store_scatter(abuf, [eidx + b], av * n16)
                cp.wait()
                pltpu.sync_copy(xrows, xe_h.at[pl.ds(base, CH)])
                pltpu.sync_copy(abuf, a_h.at[pl.ds(base * nb, CH * nb)])

            return carry

        lax.fori_loop(0, jmax, body, jnp.int32(0))

    xe, a_flat = k(table, src, etype, norm, att.reshape(r * nb))
    return xe, a_flat.reshape(e, nb)


def _sc_scatter(msg, dst, n, with_count):
    e, d = msg.shape
    nch = e // CH
    jmax = (nch + NW - 1) // NW
    nrch = n // ZR
    rjmax = (nrch + NS - 1) // NS

    out_type = [jax.ShapeDtypeStruct((NC, n, d), jnp.float32)]
    scratch = [
        pltpu.VMEM_SHARED((n, d), jnp.float32),
        pltpu.VMEM((CH,), jnp.int32),
        pltpu.VMEM((CH, d), jnp.float32),
        pltpu.VMEM((ZR, d), jnp.float32),
    ]
    if with_count:
        out_type.append(jax.ShapeDtypeStruct((NC, n, CW), jnp.float32))
        scratch += [
            pltpu.VMEM_SHARED((n, CW), jnp.float32),
            pltpu.VMEM((ZR, CW), jnp.float32),
            pltpu.VMEM((CH, CW), jnp.float32),
        ]

    @functools.partial(pl.kernel, out_type=tuple(out_type), mesh=_mesh(),
                       scratch_types=scratch,
                       compiler_params=pltpu.CompilerParams(
                           needs_layout_passes=False,
                           use_tc_tiling_on_sc=False))
    def k(msg_h, dst_h, *refs):
        if with_count:
            agg_h, cnt_h, agg_sh, didx, mbuf, zbuf, cnt_sh, zbuf2, ones = refs
        else:
            agg_h, agg_sh, didx, mbuf, zbuf = refs
        c = lax.axis_index("c")
        s = lax.axis_index("s")
        w = s * NC + c

        z16 = jnp.zeros((16,), jnp.float32)
        o16 = jnp.ones((16,), jnp.float32)

        def zfill(i, carry):
            for col in range(0, d, 16):
                zbuf[i, pl.ds(col, 16)] = z16
            if with_count:
                for col in range(0, CW, 16):
                    zbuf2[i, pl.ds(col, 16)] = z16
            return carry

        lax.fori_loop(0, ZR, zfill, jnp.int32(0))
        if with_count:
            def ofill(i, carry):
                for col in range(0, CW, 16):
                    ones[i, pl.ds(col, 16)] = o16
                return carry
            lax.fori_loop(0, CH, ofill, jnp.int32(0))

        def zero_chunks(j, carry):
            rch = s + NS * j

            @pl.when(rch < nrch)
            def _():
                rbase = rch * ZR
                pltpu.sync_copy(zbuf, agg_sh.at[pl.ds(rbase, ZR)])
                if with_count:
                    pltpu.sync_copy(zbuf2, cnt_sh.at[pl.ds(rbase, ZR)])

            return carry

        lax.fori_loop(0, rjmax, zero_chunks, jnp.int32(0))
        plsc.subcore_barrier()

        def body(j, carry):
            kk = w + NW * j

            @pl.when(kk < nch)
            def _():
                base = kk * CH
                pltpu.sync_copy(dst_h.at[pl.ds(base, CH)], didx)
                pltpu.sync_copy(msg_h.at[pl.ds(base, CH)], mbuf)
                pltpu.sync_copy(mbuf, agg_sh.at[didx], add=True)
                if with_count:
                    pltpu.sync_copy(ones, cnt_sh.at[didx], add=True)

            return carry

        lax.fori_loop(0, jmax, body, jnp.int32(0))
        plsc.subcore_barrier()

        def out_chunks(j, carry):
            rch = s + NS * j

            @pl.when(rch < nrch)
            def _():
                rbase = rch * ZR
                pltpu.sync_copy(agg_sh.at[pl.ds(rbase, ZR)],
                                agg_h.at[c, pl.ds(rbase, ZR)])
                if with_count:
                    pltpu.sync_copy(cnt_sh.at[pl.ds(rbase, ZR)],
                                    cnt_h.at[c, pl.ds(rbase, ZR)])

            return carry

        lax.fori_loop(0, rjmax, out_chunks, jnp.int32(0))

    res = k(msg, dst)
    return res if with_count else res[0]


def _tc_contract(xe, a, bmat):
    e, d = xe.shape
    nb = a.shape[1]
    be = 1600
    grid = e // be

    def body(xe_ref, a_ref, bm_ref, out_ref):
        y = jnp.dot(xe_ref[...], bm_ref[...], preferred_element_type=jnp.float32)
        av = a_ref[...]
        acc = av[:, 0:1] * y[:, 0:d]
        for b in range(1, nb):
            acc = acc + av[:, b:b + 1] * y[:, b * d:(b + 1) * d]
        out_ref[...] = acc

    return pl.pallas_call(
        body,
        grid=(grid,),
        in_specs=[
            pl.BlockSpec((be, d), lambda i: (i, 0)),
            pl.BlockSpec((be, nb), lambda i: (i, 0)),
            pl.BlockSpec(bmat.shape, lambda i: (0, 0)),
        ],
        out_specs=pl.BlockSpec((be, d), lambda i: (i, 0)),
        out_shape=jax.ShapeDtypeStruct((e, d), jnp.float32),
    )(xe, a, bmat)


def _tc_finish(agg, cnt_or_inv, x, root, bias, first_layer):
    n, d = x.shape
    bn = 2000
    grid = n // bn

    if first_layer:
        def body(agg_ref, cnt_ref, x_ref, root_ref, bias_ref, h_ref, inv_ref):
            cc = cnt_ref[0, :, 0:1] + cnt_ref[1, :, 0:1]
            inv = 1.0 / jnp.maximum(cc, 1.0)
            aggs = agg_ref[0] + agg_ref[1]
            h = aggs * inv + jnp.dot(x_ref[...], root_ref[...],
                                     preferred_element_type=jnp.float32)
            h = h + bias_ref[...]
            h_ref[...] = jnp.maximum(h, 0.0)
            inv_ref[...] = inv

        return pl.pallas_call(
            body,
            grid=(grid,),
            in_specs=[
                pl.BlockSpec((NC, bn, d), lambda i: (0, i, 0)),
                pl.BlockSpec((NC, bn, CW), lambda i: (0, i, 0)),
                pl.BlockSpec((bn, d), lambda i: (i, 0)),
                pl.BlockSpec((d, d), lambda i: (0, 0)),
                pl.BlockSpec((1, d), lambda i: (0, 0)),
            ],
            out_specs=[
                pl.BlockSpec((bn, d), lambda i: (i, 0)),
                pl.BlockSpec((bn, 1), lambda i: (i, 0)),
            ],
            out_shape=[jax.ShapeDtypeStruct((n, d), jnp.float32),
                       jax.ShapeDtypeStruct((n, 1), jnp.float32)],
        )(agg, cnt_or_inv, x, root, bias)

    def body(agg_ref, inv_ref, x_ref, root_ref, bias_ref, out_ref):
        aggs = agg_ref[0] + agg_ref[1]
        h = aggs * inv_ref[...] + jnp.dot(x_ref[...], root_ref[...],
                                          preferred_element_type=jnp.float32)
        out_ref[...] = h + bias_ref[...]

    return pl.pallas_call(
        body,
        grid=(grid,),
        in_specs=[
            pl.BlockSpec((NC, bn, d), lambda i: (0, i, 0)),
            pl.BlockSpec((bn, 1), lambda i: (i, 0)),
            pl.BlockSpec((bn, d), lambda i: (i, 0)),
            pl.BlockSpec((d, d), lambda i: (0, 0)),
            pl.BlockSpec((1, d), lambda i: (0, 0)),
        ],
        out_specs=pl.BlockSpec((bn, d), lambda i: (i, 0)),
        out_shape=jax.ShapeDtypeStruct((n, d), jnp.float32),
    )(agg, cnt_or_inv, x, root, bias)


def kernel(entity, edge_index, edge_type, edge_norm, emb_table,
           basis1, att1, root1, bias1, basis2, att2, root2, bias2):
    n, d = emb_table.shape
    nb = basis1.shape[0]
    x = emb_table
    src = edge_index[0]
    dst = edge_index[1]
    bmat1 = basis1.transpose(1, 0, 2).reshape(d, nb * d)
    bmat2 = basis2.transpose(1, 0, 2).reshape(d, nb * d)

    xe1, a1 = _sc_gather(x, src, edge_type, edge_norm, att1)
    msg1 = _tc_contract(xe1, a1, bmat1)
    agg1, cnt = _sc_scatter(msg1, dst, n, with_count=True)
    h, inv = _tc_finish(agg1, cnt, x, root1, bias1.reshape(1, d), first_layer=True)

    xe2, a2 = _sc_gather(h, src, edge_type, edge_norm, att2)
    msg2 = _tc_contract(xe2, a2, bmat2)
    agg2 = _sc_scatter(msg2, dst, n, with_count=False)
    out = _tc_finish(agg2, inv, h, root2, bias2.reshape(1, d), first_layer=False)
    return out


# speedup vs baseline: 3.2665x; 1.6549x over previous
import functools

import jax
import jax.numpy as jnp
from jax import lax
from jax.experimental import pallas as pl
from jax.experimental.pallas import tpu as pltpu
from jax.experimental.pallas import tpu_sc as plsc

NC = 2
NS = 16
NW = NC * NS
CH = 128
CW = 8
ZR = 160


def _mesh():
    return plsc.VectorSubcoreMesh(core_axis_name="c", subcore_axis_name="s",
                                  num_cores=NC, num_subcores=NS)


def _sc_gather(table, src, etype, norm, att):
    n, d = table.shape
    e = src.shape[0]
    r, nb = att.shape
    nch = e // CH
    jmax = (nch + NW - 1) // NW

    @functools.partial(
        pl.kernel,
        out_type=(jax.ShapeDtypeStruct((e, d), jnp.float32),
                  jax.ShapeDtypeStruct((e, nb), jnp.float32)),
        mesh=_mesh(),
        scratch_types=[
            pltpu.VMEM((r * nb,), jnp.float32),
            pltpu.VMEM((CH,), jnp.int32),
            pltpu.VMEM((CH,), jnp.int32),
            pltpu.VMEM((CH,), jnp.float32),
            pltpu.VMEM((CH, d), jnp.float32),
            pltpu.VMEM((CH, nb), jnp.float32),
            pltpu.SemaphoreType.DMA,
        ],
        compiler_params=pltpu.CompilerParams(needs_layout_passes=False,
                                             use_tc_tiling_on_sc=False),
    )
    def k(table_h, src_h, et_h, norm_h, att_h, xe_h, a_h,
          att_v, sidx, tbuf, nbuf, xrows, abuf, sem):
        c = lax.axis_index("c")
        s = lax.axis_index("s")
        w = s * NC + c
        pltpu.sync_copy(att_h, att_v)

        def body(j, carry):
            kk = w + NW * j

            @pl.when(kk < nch)
            def _():
                base = kk * CH
                pltpu.sync_copy(src_h.at[pl.ds(base, CH)], sidx)
                cp = pltpu.async_copy(table_h.at[sidx], xrows, sem)
                pltpu.sync_copy(et_h.at[pl.ds(base, CH)], tbuf)
                pltpu.sync_copy(norm_h.at[pl.ds(base, CH)], nbuf)
                for g in range(CH // 16):
                    t16 = tbuf[pl.ds(g * 16, 16)] * nb
                    n16 = nbuf[pl.ds(g * 16, 16)]
                    eidx = lax.iota(jnp.int32, 16) + g * 16
                    for b in range(nb):
                        bfull = jnp.full((16,), b, jnp.int32)
                        av = plsc.load_gather(att_v, [t16 + b])
                        plsc.store_scatter(abuf, [eidx, bfull], av * n16)
                cp.wait()
                pltpu.sync_copy(xrows, xe_h.at[pl.ds(base, CH)])
                pltpu.sync_copy(abuf, a_h.at[pl.ds(base, CH)])

            return carry

        lax.fori_loop(0, jmax, body, jnp.int32(0))

    return k(table, src, etype, norm, att.reshape(r * nb))


def _sc_scatter(msg, dst, n, with_count):
    e, d = msg.shape
    nch = e // CH
    jmax = (nch + NW - 1) // NW
    nrch = n // ZR
    rjmax = (nrch + NS - 1) // NS

    out_type = [jax.ShapeDtypeStruct((NC, n, d), jnp.float32)]
    scratch = [
        pltpu.VMEM_SHARED((n, d), jnp.float32),
        pltpu.VMEM((CH,), jnp.int32),
        pltpu.VMEM((CH, d), jnp.float32),
        pltpu.VMEM((ZR, d), jnp.float32),
    ]
    if with_count:
        out_type.append(jax.ShapeDtypeStruct((NC, n, CW), jnp.float32))
        scratch += [
            pltpu.VMEM_SHARED((n, CW), jnp.float32),
            pltpu.VMEM((ZR, CW), jnp.float32),
            pltpu.VMEM((CH, CW), jnp.float32),
        ]

    @functools.partial(pl.kernel, out_type=tuple(out_type), mesh=_mesh(),
                       scratch_types=scratch,
                       compiler_params=pltpu.CompilerParams(
                           needs_layout_passes=False,
                           use_tc_tiling_on_sc=False))
    def k(msg_h, dst_h, *refs):
        if with_count:
            agg_h, cnt_h, agg_sh, didx, mbuf, zbuf, cnt_sh, zbuf2, ones = refs
        else:
            agg_h, agg_sh, didx, mbuf, zbuf = refs
        c = lax.axis_index("c")
        s = lax.axis_index("s")
        w = s * NC + c

        z16 = jnp.zeros((16,), jnp.float32)
        o16 = jnp.ones((16,), jnp.float32)

        def zfill(i, carry):
            for col in range(0, d, 16):
                zbuf[i, pl.ds(col, 16)] = z16
            if with_count:
                for col in range(0, CW, 16):
                    zbuf2[i, pl.ds(col, 16)] = z16
            return carry

        lax.fori_loop(0, ZR, zfill, jnp.int32(0))
        if with_count:
            def ofill(i, carry):
                for col in range(0, CW, 16):
                    ones[i, pl.ds(col, 16)] = o16
                return carry
            lax.fori_loop(0, CH, ofill, jnp.int32(0))

        def zero_chunks(j, carry):
            rch = s + NS * j

            @pl.when(rch < nrch)
            def _():
                rbase = rch * ZR
                pltpu.sync_copy(zbuf, agg_sh.at[pl.ds(rbase, ZR)])
                if with_count:
                    pltpu.sync_copy(zbuf2, cnt_sh.at[pl.ds(rbase, ZR)])

            return carry

        lax.fori_loop(0, rjmax, zero_chunks, jnp.int32(0))
        plsc.subcore_barrier()

        def body(j, carry):
            kk = w + NW * j

            @pl.when(kk < nch)
            def _():
                base = kk * CH
                pltpu.sync_copy(dst_h.at[pl.ds(base, CH)], didx)
                pltpu.sync_copy(msg_h.at[pl.ds(base, CH)], mbuf)
                pltpu.sync_copy(mbuf, agg_sh.at[didx], add=True)
                if with_count:
                    pltpu.sync_copy(ones, cnt_sh.at[didx], add=True)

            return carry

        lax.fori_loop(0, jmax, body, jnp.int32(0))
        plsc.subcore_barrier()

        def out_chunks(j, carry):
            rch = s + NS * j

            @pl.when(rch < nrch)
            def _():
                rbase = rch * ZR
                pltpu.sync_copy(agg_sh.at[pl.ds(rbase, ZR)],
                                agg_h.at[c, pl.ds(rbase, ZR)])
                if with_count:
                    pltpu.sync_copy(cnt_sh.at[pl.ds(rbase, ZR)],
                                    cnt_h.at[c, pl.ds(rbase, ZR)])

            return carry

        lax.fori_loop(0, rjmax, out_chunks, jnp.int32(0))

    res = k(msg, dst)
    return res if with_count else res[0]


def _tc_contract(xe, a, bmat, tmat, smat):
    e, d = xe.shape
    nb = a.shape[1]
    be = 1600
    grid = e // be

    def body(xe_ref, a_ref, bm_ref, t_ref, s_ref, out_ref):
        y = jnp.dot(xe_ref[...], bm_ref[...], preferred_element_type=jnp.float32)
        at = jnp.dot(a_ref[...], t_ref[...], preferred_element_type=jnp.float32)
        out_ref[...] = jnp.dot(at * y, s_ref[...],
                               preferred_element_type=jnp.float32)

    return pl.pallas_call(
        body,
        grid=(grid,),
        in_specs=[
            pl.BlockSpec((be, d), lambda i: (i, 0)),
            pl.BlockSpec((be, nb), lambda i: (i, 0)),
            pl.BlockSpec(bmat.shape, lambda i: (0, 0)),
            pl.BlockSpec(tmat.shape, lambda i: (0, 0)),
            pl.BlockSpec(smat.shape, lambda i: (0, 0)),
        ],
        out_specs=pl.BlockSpec((be, d), lambda i: (i, 0)),
        out_shape=jax.ShapeDtypeStruct((e, d), jnp.float32),
    )(xe, a, bmat, tmat, smat)


def _tc_finish(agg, cnt_or_inv, x, root, bias, first_layer):
    n, d = x.shape
    bn = 2000
    grid = n // bn

    if first_layer:
        def body(agg_ref, cnt_ref, x_ref, root_ref, bias_ref, h_ref, inv_ref):
            cc = cnt_ref[0, :, 0:1] + cnt_ref[1, :, 0:1]
            inv = 1.0 / jnp.maximum(cc, 1.0)
            aggs = agg_ref[0] + agg_ref[1]
            h = aggs * inv + jnp.dot(x_ref[...], root_ref[...],
                                     preferred_element_type=jnp.float32)
            h = h + bias_ref[...]
            h_ref[...] = jnp.maximum(h, 0.0)
            inv_ref[...] = inv

        return pl.pallas_call(
            body,
            grid=(grid,),
            in_specs=[
                pl.BlockSpec((NC, bn, d), lambda i: (0, i, 0)),
                pl.BlockSpec((NC, bn, CW), lambda i: (0, i, 0)),
                pl.BlockSpec((bn, d), lambda i: (i, 0)),
                pl.BlockSpec((d, d), lambda i: (0, 0)),
                pl.BlockSpec((1, d), lambda i: (0, 0)),
            ],
            out_specs=[
                pl.BlockSpec((bn, d), lambda i: (i, 0)),
                pl.BlockSpec((bn, 1), lambda i: (i, 0)),
            ],
            out_shape=[jax.ShapeDtypeStruct((n, d), jnp.float32),
                       jax.ShapeDtypeStruct((n, 1), jnp.float32)],
        )(agg, cnt_or_inv, x, root, bias)

    def body(agg_ref, inv_ref, x_ref, root_ref, bias_ref, out_ref):
        aggs = agg_ref[0] + agg_ref[1]
        h = aggs * inv_ref[...] + jnp.dot(x_ref[...], root_ref[...],
                                          preferred_element_type=jnp.float32)
        out_ref[...] = h + bias_ref[...]

    return pl.pallas_call(
        body,
        grid=(grid,),
        in_specs=[
            pl.BlockSpec((NC, bn, d), lambda i: (0, i, 0)),
            pl.BlockSpec((bn, 1), lambda i: (i, 0)),
            pl.BlockSpec((bn, d), lambda i: (i, 0)),
            pl.BlockSpec((d, d), lambda i: (0, 0)),
            pl.BlockSpec((1, d), lambda i: (0, 0)),
        ],
        out_specs=pl.BlockSpec((bn, d), lambda i: (i, 0)),
        out_shape=jax.ShapeDtypeStruct((n, d), jnp.float32),
    )(agg, cnt_or_inv, x, root, bias)


def kernel(entity, edge_index, edge_type, edge_norm, emb_table,
           basis1, att1, root1, bias1, basis2, att2, root2, bias2):
    n, d = emb_table.shape
    nb = basis1.shape[0]
    x = emb_table
    src = edge_index[0]
    dst = edge_index[1]
    bmat1 = basis1.transpose(1, 2, 0).reshape(d, d * nb)
    bmat2 = basis2.transpose(1, 2, 0).reshape(d, d * nb)
    tmat = jnp.tile(jnp.eye(nb, dtype=jnp.float32), (1, d))
    smat = jnp.repeat(jnp.eye(d, dtype=jnp.float32), nb, axis=0)

    xe1, a1 = _sc_gather(x, src, edge_type, edge_norm, att1)
    msg1 = _tc_contract(xe1, a1, bmat1, tmat, smat)
    agg1, cnt = _sc_scatter(msg1, dst, n, with_count=True)
    h, inv = _tc_finish(agg1, cnt, x, root1, bias1.reshape(1, d), first_layer=True)

    xe2, a2 = _sc_gather(h, src, edge_type, edge_norm, att2)
    msg2 = _tc_contract(xe2, a2, bmat2, tmat, smat)
    agg2 = _sc_scatter(msg2, dst, n, with_count=False)
    out = _tc_finish(agg2, inv, h, root2, bias2.reshape(1, d), first_layer=False)
    return out
